# baseline (speedup 1.0000x reference)
import functools

import numpy as np
import jax
import jax.numpy as jnp
from jax import lax
from jax.experimental import pallas as pl
from jax.experimental.pallas import tpu as pltpu
from jax.experimental.pallas import tpu_sc as plsc

_NB = 18
_NCOL = 9
_F = 8
_T = 1024
_NW = 32
_ROWS = _F * _NCOL * _NB
_CHUNKS = _T // 16

def _sc_histogram(phat, ampt, cuts):
    mesh = plsc.VectorSubcoreMesh(core_axis_name="c", subcore_axis_name="s")

    @functools.partial(
        pl.kernel,
        out_type=jax.ShapeDtypeStruct((_NW, _ROWS * 16), jnp.float32),
        mesh=mesh,
        compiler_params=pltpu.CompilerParams(needs_layout_passes=False),
        scratch_types=[
            pltpu.VMEM((_F, _T), jnp.float32),
            pltpu.VMEM((_F, _T), jnp.float32),
            pltpu.VMEM((_ROWS * 16,), jnp.float32),
            pltpu.VMEM(((_NB + 1) * 16,), jnp.float32),
        ],
    )
    def k(pha_hbm, amp_hbm, cut_hbm, out_hbm, pha_v, amp_v, hist_v, cut_v):
        wid = lax.axis_index("s") * 2 + lax.axis_index("c")
        ci = wid // 4
        si = wid - ci * 4
        pltpu.sync_copy(pha_hbm.at[0, ci, :, si, :], pha_v)
        pltpu.sync_copy(amp_hbm.at[0, ci, :, si, :], amp_v)
        pltpu.sync_copy(cut_hbm, cut_v)

        zero16 = jnp.zeros((16,), jnp.float32)
        for r in range(_ROWS):
            hist_v[pl.ds(r * 16, 16)] = zero16

        ones16 = jnp.ones((16,), jnp.float32)
        lane = lax.iota(jnp.int32, 16)
        pi_f = jnp.float32(np.pi)
        inv_w = jnp.float32(_NB / (2.0 * np.pi))
        zero_i = jnp.zeros((16,), jnp.int32)
        hi_i = jnp.full((16,), _NB - 1, jnp.int32)

        def chunk_body(t, carry):
            off = t * 16
            amps = [amp_v[fa, pl.ds(off, 16)] for fa in range(_F)]
            for fp in range(_F):
                x = pha_v[fp, pl.ds(off, 16)]
                b0 = ((x + pi_f) * inv_w).astype(jnp.int32)
                b0 = jnp.minimum(jnp.maximum(b0, zero_i), hi_i)
                c_hi = plsc.load_gather(cut_v, [b0 * 16 + (lane + 16)])
                u = jnp.where(x > c_hi, b0 + 1, b0)
                c_u = plsc.load_gather(cut_v, [u * 16 + lane])
                d = jnp.where(x > c_u, u, u - 1)
                b = jnp.minimum(jnp.maximum(d, zero_i), hi_i)
                base = b * 128 + (lane + fp * 16)
                for fa in range(_F):
                    plsc.addupdate_scatter(hist_v, [base + fa * (_NB * 128)], amps[fa])
                plsc.addupdate_scatter(hist_v, [base + _F * (_NB * 128)], ones16)
            return carry

        lax.fori_loop(0, _CHUNKS, chunk_body, 0)
        pltpu.sync_copy(hist_v, out_hbm.at[wid])

    return k(phat, ampt, cuts)


def _mi_body(h_ref, o_ref):
    eps = jnp.float32(1e-9)
    h = h_ref[...]
    hm = h.reshape(_NW * (_NCOL * _NB), 128)
    li = jax.lax.broadcasted_iota(jnp.int32, (128, _F), 0)
    gi = jax.lax.broadcasted_iota(jnp.int32, (128, _F), 1)
    G = (li // 16 == gi).astype(jnp.float32)
    s8 = jax.lax.dot_general(hm, G, (((1,), (0,)), ((), ())),
                             precision=jax.lax.Precision.HIGHEST)
    s8 = s8.reshape(_NW, _NCOL * _NB, _F)
    counts = s8[:, _F * _NB:, :]
    wi = jax.lax.broadcasted_iota(jnp.int32, (_F, _NW), 1)
    ci = jax.lax.broadcasted_iota(jnp.int32, (_F, _NW), 0)
    A = jnp.where(wi // 4 == ci, jnp.float32(0.25), jnp.float32(0.0))
    nb = jnp.float32(_NB)
    for fa in range(_F):
        sums = s8[:, fa * _NB:(fa + 1) * _NB, :]
        means = sums / (counts + eps)
        probs = means / (jnp.sum(means, axis=1, keepdims=True) + eps)
        ent = jnp.sum(probs * jnp.log(probs + eps), axis=1)
        mi = (jnp.log(nb + eps) + ent) / jnp.log(nb)
        o_ref[fa] = jax.lax.dot_general(
            A, mi, (((1,), (0,)), ((), ())),
            precision=jax.lax.Precision.HIGHEST)


def kernel(pha, amp):
    B, C, F, S, T = pha.shape
    cutoffs = jnp.linspace(-np.pi, np.pi, _NB + 1).astype(jnp.float32)
    cuts = jnp.tile(cutoffs[:, None], (1, 16)).reshape((_NB + 1) * 16)
    hist = _sc_histogram(pha, amp, cuts)
    h3 = hist.reshape(_NW, _NCOL * _NB, 128)

    mi = pl.pallas_call(
        _mi_body,
        out_shape=jax.ShapeDtypeStruct((F, C, F), jnp.float32),
    )(h3)
    return jnp.transpose(mi, (1, 2, 0)).reshape(B, C, F, F)


# speedup vs baseline: 1.0659x; 1.0659x over previous
import functools

import numpy as np
import jax
import jax.numpy as jnp
from jax import lax
from jax.experimental import pallas as pl
from jax.experimental.pallas import tpu as pltpu
from jax.experimental.pallas import tpu_sc as plsc

_NB = 18
_NCOL = 9
_F = 8
_T = 1024
_NW = 32
_ROWS = _F * _NCOL * _NB
_CHUNKS = _T // 16

def _sc_histogram(phat, ampt, cuts):
    mesh = plsc.VectorSubcoreMesh(core_axis_name="c", subcore_axis_name="s")

    @functools.partial(
        pl.kernel,
        out_type=jax.ShapeDtypeStruct((_NW, _ROWS * 16), jnp.float32),
        mesh=mesh,
        compiler_params=pltpu.CompilerParams(needs_layout_passes=False),
        scratch_types=[
            pltpu.VMEM((_F, _T), jnp.float32),
            pltpu.VMEM((_F, _T), jnp.float32),
            pltpu.VMEM((_ROWS * 16,), jnp.float32),
            pltpu.VMEM(((_NB + 1) * 16,), jnp.float32),
        ],
    )
    def k(pha_hbm, amp_hbm, cut_hbm, out_hbm, pha_v, amp_v, hist_v, cut_v):
        wid = lax.axis_index("s") * 2 + lax.axis_index("c")
        ci = wid // 4
        si = wid - ci * 4
        pltpu.sync_copy(pha_hbm.at[0, ci, :, si, :], pha_v)
        pltpu.sync_copy(amp_hbm.at[0, ci, :, si, :], amp_v)
        pltpu.sync_copy(cut_hbm, cut_v)

        zero16 = jnp.zeros((16,), jnp.float32)
        for r in range(_ROWS):
            hist_v[pl.ds(r * 16, 16)] = zero16

        ones16 = jnp.ones((16,), jnp.float32)
        lane = lax.iota(jnp.int32, 16)
        pi_f = jnp.float32(np.pi)
        inv_w = jnp.float32(_NB / (2.0 * np.pi))
        zero_i = jnp.zeros((16,), jnp.int32)
        hi_i = jnp.full((16,), _NB - 1, jnp.int32)

        def chunk_body(t, carry):
            off = t * 16
            amps = [amp_v[fa, pl.ds(off, 16)] for fa in range(_F)]
            for fp in range(_F):
                x = pha_v[fp, pl.ds(off, 16)]
                b0 = ((x + pi_f) * inv_w).astype(jnp.int32)
                b0 = jnp.minimum(jnp.maximum(b0, zero_i), hi_i)
                g_lo = b0 * 16 + lane
                c_lo = plsc.load_gather(cut_v, [g_lo])
                c_hi = plsc.load_gather(cut_v, [g_lo + 16])
                d = b0 + jnp.where(x > c_hi, 1, 0) - jnp.where(x > c_lo, 0, 1)
                b = jnp.minimum(jnp.maximum(d, zero_i), hi_i)
                base = b * 128 + (lane + fp * 16)
                for fa in range(_F):
                    plsc.addupdate_scatter(hist_v, [base + fa * (_NB * 128)], amps[fa])
                plsc.addupdate_scatter(hist_v, [base + _F * (_NB * 128)], ones16)
            return carry

        lax.fori_loop(0, _CHUNKS, chunk_body, 0)
        pltpu.sync_copy(hist_v, out_hbm.at[wid])

    return k(phat, ampt, cuts)


def _mi_body(h_ref, o_ref):
    eps = jnp.float32(1e-9)
    h = h_ref[...]
    hm = h.reshape(_NW * (_NCOL * _NB), 128)
    li = jax.lax.broadcasted_iota(jnp.int32, (128, _F), 0)
    gi = jax.lax.broadcasted_iota(jnp.int32, (128, _F), 1)
    G = (li // 16 == gi).astype(jnp.float32)
    s8 = jax.lax.dot_general(hm, G, (((1,), (0,)), ((), ())),
                             precision=jax.lax.Precision.HIGHEST)
    s8 = s8.reshape(_NW, _NCOL * _NB, _F)
    counts = s8[:, _F * _NB:, :]
    wi = jax.lax.broadcasted_iota(jnp.int32, (_F, _NW), 1)
    ci = jax.lax.broadcasted_iota(jnp.int32, (_F, _NW), 0)
    A = jnp.where(wi // 4 == ci, jnp.float32(0.25), jnp.float32(0.0))
    nb = jnp.float32(_NB)
    for fa in range(_F):
        sums = s8[:, fa * _NB:(fa + 1) * _NB, :]
        means = sums / (counts + eps)
        probs = means / (jnp.sum(means, axis=1, keepdims=True) + eps)
        ent = jnp.sum(probs * jnp.log(probs + eps), axis=1)
        mi = (jnp.log(nb + eps) + ent) / jnp.log(nb)
        o_ref[fa] = jax.lax.dot_general(
            A, mi, (((1,), (0,)), ((), ())),
            precision=jax.lax.Precision.HIGHEST)


def kernel(pha, amp):
    B, C, F, S, T = pha.shape
    cutoffs = jnp.linspace(-np.pi, np.pi, _NB + 1).astype(jnp.float32)
    cuts = jnp.tile(cutoffs[:, None], (1, 16)).reshape((_NB + 1) * 16)
    hist = _sc_histogram(pha, amp, cuts)
    h3 = hist.reshape(_NW, _NCOL * _NB, 128)

    mi = pl.pallas_call(
        _mi_body,
        out_shape=jax.ShapeDtypeStruct((F, C, F), jnp.float32),
    )(h3)
    return jnp.transpose(mi, (1, 2, 0)).reshape(B, C, F, F)


# speedup vs baseline: 1.1430x; 1.0724x over previous
import functools

import numpy as np
import jax
import jax.numpy as jnp
from jax import lax
from jax.experimental import pallas as pl
from jax.experimental.pallas import tpu as pltpu
from jax.experimental.pallas import tpu_sc as plsc

_NB = 18
_NCOL = 9
_F = 8
_T = 1024
_NW = 32
_ROWS = _F * _NCOL * _NB
_CHUNKS = _T // 16

_CUTS = [float(v) for v in np.linspace(-np.pi, np.pi, _NB + 1).astype(np.float32)[1:_NB]]


def _sc_histogram(phat, ampt):
    mesh = plsc.VectorSubcoreMesh(core_axis_name="c", subcore_axis_name="s")

    @functools.partial(
        pl.kernel,
        out_type=jax.ShapeDtypeStruct((_NW, _ROWS * 16), jnp.float32),
        mesh=mesh,
        compiler_params=pltpu.CompilerParams(needs_layout_passes=False),
        scratch_types=[
            pltpu.VMEM((_F, _T), jnp.float32),
            pltpu.VMEM((_F, _T), jnp.float32),
            pltpu.VMEM((_ROWS * 16,), jnp.float32),
        ],
    )
    def k(pha_hbm, amp_hbm, out_hbm, pha_v, amp_v, hist_v):
        wid = lax.axis_index("s") * 2 + lax.axis_index("c")
        ci = wid // 4
        si = wid - ci * 4
        pltpu.sync_copy(pha_hbm.at[0, ci, :, si, :], pha_v)
        pltpu.sync_copy(amp_hbm.at[0, ci, :, si, :], amp_v)

        zero16 = jnp.zeros((16,), jnp.float32)
        for r in range(_ROWS):
            hist_v[pl.ds(r * 16, 16)] = zero16

        ones16 = jnp.ones((16,), jnp.float32)
        lane = lax.iota(jnp.int32, 16)
        zero_i = jnp.zeros((16,), jnp.int32)
        one_i = jnp.ones((16,), jnp.int32)

        def chunk_body(t, carry):
            off = t * 16
            amps = [amp_v[fa, pl.ds(off, 16)] for fa in range(_F)]
            for fp in range(_F):
                x = pha_v[fp, pl.ds(off, 16)]
                ms = [jnp.where(x > c, one_i, zero_i) for c in _CUTS]
                while len(ms) > 1:
                    ms = [ms[i] + ms[i + 1] for i in range(0, len(ms) - 1, 2)] + (
                        [ms[-1]] if len(ms) % 2 else [])
                b = ms[0]
                base = b * 128 + (lane + fp * 16)
                for fa in range(_F):
                    plsc.addupdate_scatter(hist_v, [base + fa * (_NB * 128)], amps[fa])
                plsc.addupdate_scatter(hist_v, [base + _F * (_NB * 128)], ones16)
            return carry

        lax.fori_loop(0, _CHUNKS, chunk_body, 0)
        pltpu.sync_copy(hist_v, out_hbm.at[wid])

    return k(phat, ampt)


def _mi_body(h_ref, o_ref):
    eps = jnp.float32(1e-9)
    h = h_ref[...]
    hm = h.reshape(_NW * (_NCOL * _NB), 128)
    li = jax.lax.broadcasted_iota(jnp.int32, (128, _F), 0)
    gi = jax.lax.broadcasted_iota(jnp.int32, (128, _F), 1)
    G = (li // 16 == gi).astype(jnp.float32)
    s8 = jax.lax.dot_general(hm, G, (((1,), (0,)), ((), ())),
                             precision=jax.lax.Precision.HIGHEST)
    s8 = s8.reshape(_NW, _NCOL * _NB, _F)
    counts = s8[:, _F * _NB:, :]
    wi = jax.lax.broadcasted_iota(jnp.int32, (_F, _NW), 1)
    ci = jax.lax.broadcasted_iota(jnp.int32, (_F, _NW), 0)
    A = jnp.where(wi // 4 == ci, jnp.float32(0.25), jnp.float32(0.0))
    nb = jnp.float32(_NB)
    for fa in range(_F):
        sums = s8[:, fa * _NB:(fa + 1) * _NB, :]
        means = sums / (counts + eps)
        probs = means / (jnp.sum(means, axis=1, keepdims=True) + eps)
        ent = jnp.sum(probs * jnp.log(probs + eps), axis=1)
        mi = (jnp.log(nb + eps) + ent) / jnp.log(nb)
        o_ref[fa] = jax.lax.dot_general(
            A, mi, (((1,), (0,)), ((), ())),
            precision=jax.lax.Precision.HIGHEST)


def kernel(pha, amp):
    B, C, F, S, T = pha.shape
    hist = _sc_histogram(pha, amp)
    h3 = hist.reshape(_NW, _NCOL * _NB, 128)

    mi = pl.pallas_call(
        _mi_body,
        out_shape=jax.ShapeDtypeStruct((F, C, F), jnp.float32),
    )(h3)
    return jnp.transpose(mi, (1, 2, 0)).reshape(B, C, F, F)
